# baseline (device time: 58129 ns/iter reference)
import jax
import jax.numpy as jnp
from jax import lax
from jax.experimental import pallas as pl
from jax.experimental.pallas import tpu as pltpu

HALF_ROWS = 8


def kernel(Q, K, V):
    b, s, h, d = Q.shape
    hb = h * b
    scale = d ** -0.5

    def to_rows(A):
        return jnp.transpose(A, (2, 0, 1, 3)).reshape(hb, s, d)

    def body(q_ref, k_ref, v_ref, out_ref, kv_recv, send_sems, recv_sems):
        my_x = lax.axis_index("x")
        my_y = lax.axis_index("y")
        y_nbr = (my_x, 1 - my_y)
        x_nbr = (1 - my_x, my_y)

        barrier = pltpu.get_barrier_semaphore()
        for nbr in (y_nbr, x_nbr):
            pl.semaphore_signal(
                barrier, inc=1, device_id=nbr,
                device_id_type=pl.DeviceIdType.MESH,
            )
        pl.semaphore_wait(barrier, 2)

        base = my_x * HALF_ROWS

        k_rdma = pltpu.make_async_remote_copy(
            src_ref=k_ref.at[pl.ds(base, HALF_ROWS)],
            dst_ref=kv_recv.at[0],
            send_sem=send_sems.at[0], recv_sem=recv_sems.at[0],
            device_id=y_nbr, device_id_type=pl.DeviceIdType.MESH,
        )
        v_rdma = pltpu.make_async_remote_copy(
            src_ref=v_ref.at[pl.ds(base, HALF_ROWS)],
            dst_ref=kv_recv.at[1],
            send_sem=send_sems.at[1], recv_sem=recv_sems.at[1],
            device_id=y_nbr, device_id_type=pl.DeviceIdType.MESH,
        )
        k_rdma.start()
        v_rdma.start()
        k_rdma.wait()
        v_rdma.wait()

        qh = q_ref[pl.ds(base, HALF_ROWS)]
        out_ref[pl.ds(0, HALF_ROWS)] = kv_recv[0]
        out_ref[pl.ds(HALF_ROWS, HALF_ROWS)] = kv_recv[1]
        kr = None


    out_rows = pl.pallas_call(
        body,
        out_shape=jax.ShapeDtypeStruct((hb, s, d), jnp.float32),
        in_specs=[pl.BlockSpec(memory_space=pltpu.VMEM)] * 3,
        out_specs=pl.BlockSpec(memory_space=pltpu.VMEM),
        scratch_shapes=[
            pltpu.VMEM((2, HALF_ROWS, s, d), jnp.float32),
            pltpu.SemaphoreType.DMA((3,)),
            pltpu.SemaphoreType.DMA((3,)),
        ],
        compiler_params=pltpu.CompilerParams(collective_id=0),
    )(to_rows(Q), to_rows(K), to_rows(V))

    return jnp.transpose(out_rows.reshape(h, b, s, d), (1, 2, 0, 3))


# device time: 42505 ns/iter; 1.3676x vs baseline; 1.3676x over previous
import jax
import jax.numpy as jnp
from jax import lax
from jax.experimental import pallas as pl
from jax.experimental.pallas import tpu as pltpu

HALF_ROWS = 8
NCHUNK = 4
RPC = HALF_ROWS // NCHUNK


def kernel(Q, K, V):
    b, s, h, d = Q.shape
    hb = h * b
    scale = d ** -0.5

    def to_rows(A):
        return jnp.transpose(A, (2, 0, 1, 3)).reshape(hb, s, d)

    def body(q_ref, k_ref, v_ref, out_ref, kv_send, kv_recv, o_send, o_recv,
             kv_ssem, kv_rsem, o_ssem, o_rsem):
        my_x = lax.axis_index("x")
        my_y = lax.axis_index("y")
        y_nbr = (my_x, 1 - my_y)
        x_nbr = (1 - my_x, my_y)

        barrier = pltpu.get_barrier_semaphore()
        for nbr in (y_nbr, x_nbr):
            pl.semaphore_signal(
                barrier, inc=1, device_id=nbr,
                device_id_type=pl.DeviceIdType.MESH,
            )
        pl.semaphore_wait(barrier, 2)

        base = my_x * HALF_ROWS
        other = (1 - my_x) * HALF_ROWS

        qh = q_ref[pl.ds(base, HALF_ROWS)].astype(jnp.bfloat16)
        kh = k_ref[pl.ds(base, HALF_ROWS)].astype(jnp.bfloat16)
        vh = v_ref[pl.ds(base, HALF_ROWS)].astype(jnp.bfloat16)

        kv_rdmas = []
        for c in range(NCHUNK):
            kv_send[c, 0] = kh[2 * c:2 * c + RPC]
            kv_send[c, 1] = vh[2 * c:2 * c + RPC]
            r = pltpu.make_async_remote_copy(
                src_ref=kv_send.at[c], dst_ref=kv_recv.at[c],
                send_sem=kv_ssem.at[c], recv_sem=kv_rsem.at[c],
                device_id=y_nbr, device_id_type=pl.DeviceIdType.MESH,
            )
            r.start()
            kv_rdmas.append(r)

        o_rdmas = []
        for c in range(NCHUNK):
            kv_rdmas[c].wait_recv()
            outs = []
            for j in range(RPC):
                i = 2 * c + j
                k_all = jnp.concatenate([kh[i], kv_recv[c, 0, j]], axis=0)
                v_all = jnp.concatenate([vh[i], kv_recv[c, 1, j]], axis=0)
                s_i = lax.dot_general(
                    qh[i], k_all, (((1,), (1,)), ((), ())),
                    preferred_element_type=jnp.float32,
                ) * scale
                m = jnp.max(s_i, axis=1, keepdims=True)
                p = jnp.exp(s_i - m)
                l = jnp.sum(p, axis=1, keepdims=True)
                o_i = lax.dot_general(
                    (p / l).astype(jnp.bfloat16), v_all,
                    (((1,), (0,)), ((), ())),
                    preferred_element_type=jnp.float32,
                )
                outs.append(o_i[None])
                o_send[c, j] = o_i.astype(jnp.bfloat16)
            out_ref[pl.ds(base + 2 * c, RPC)] = jnp.concatenate(outs, axis=0)
            r = pltpu.make_async_remote_copy(
                src_ref=o_send.at[c], dst_ref=o_recv.at[c],
                send_sem=o_ssem.at[c], recv_sem=o_rsem.at[c],
                device_id=x_nbr, device_id_type=pl.DeviceIdType.MESH,
            )
            r.start()
            o_rdmas.append(r)

        for c in range(NCHUNK):
            o_rdmas[c].wait_recv()
            out_ref[pl.ds(other + 2 * c, RPC)] = o_recv[c].astype(jnp.float32)

        for c in range(NCHUNK):
            kv_rdmas[c].wait_send()
            o_rdmas[c].wait_send()

    out_rows = pl.pallas_call(
        body,
        out_shape=jax.ShapeDtypeStruct((hb, s, d), jnp.float32),
        in_specs=[pl.BlockSpec(memory_space=pltpu.VMEM)] * 3,
        out_specs=pl.BlockSpec(memory_space=pltpu.VMEM),
        scratch_shapes=[
            pltpu.VMEM((NCHUNK, 2, RPC, s, d), jnp.bfloat16),
            pltpu.VMEM((NCHUNK, 2, RPC, s, d), jnp.bfloat16),
            pltpu.VMEM((NCHUNK, RPC, s, d), jnp.bfloat16),
            pltpu.VMEM((NCHUNK, RPC, s, d), jnp.bfloat16),
            pltpu.SemaphoreType.DMA((NCHUNK,)),
            pltpu.SemaphoreType.DMA((NCHUNK,)),
            pltpu.SemaphoreType.DMA((NCHUNK,)),
            pltpu.SemaphoreType.DMA((NCHUNK,)),
        ],
        compiler_params=pltpu.CompilerParams(collective_id=0),
    )(to_rows(Q), to_rows(K), to_rows(V))

    return jnp.transpose(out_rows.reshape(h, b, s, d), (1, 2, 0, 3))


# device time: 41170 ns/iter; 1.4119x vs baseline; 1.0324x over previous
import jax
import jax.numpy as jnp
from jax import lax
from jax.experimental import pallas as pl
from jax.experimental.pallas import tpu as pltpu

HALF_ROWS = 8
NCHUNK = 4
RPC = HALF_ROWS // NCHUNK


def kernel(Q, K, V):
    b, s, h, d = Q.shape
    hb = h * b
    scale = d ** -0.5

    def to_rows(A):
        return jnp.transpose(A, (2, 0, 1, 3)).reshape(hb, s, d)

    def body(q_ref, k_ref, v_ref, out_ref, kv_send, kv_recv, o_send, o_recv,
             kv_ssem, kv_rsem, o_ssem, o_rsem):
        my_x = lax.axis_index("x")
        my_y = lax.axis_index("y")
        y_nbr = (my_x, 1 - my_y)
        x_nbr = (1 - my_x, my_y)

        barrier = pltpu.get_barrier_semaphore()
        for nbr in (y_nbr, x_nbr):
            pl.semaphore_signal(
                barrier, inc=1, device_id=nbr,
                device_id_type=pl.DeviceIdType.MESH,
            )
        pl.semaphore_wait(barrier, 2)

        base = my_x * HALF_ROWS
        other = (1 - my_x) * HALF_ROWS

        kv_rdmas = []
        kh = []
        vh = []
        for c in range(NCHUNK):
            kc = k_ref[pl.ds(base + 2 * c, RPC)].astype(jnp.bfloat16)
            vc = v_ref[pl.ds(base + 2 * c, RPC)].astype(jnp.bfloat16)
            kh.append(kc)
            vh.append(vc)
            kv_send[c, 0] = kc
            kv_send[c, 1] = vc
            r = pltpu.make_async_remote_copy(
                src_ref=kv_send.at[c], dst_ref=kv_recv.at[c],
                send_sem=kv_ssem.at[c], recv_sem=kv_rsem.at[c],
                device_id=y_nbr, device_id_type=pl.DeviceIdType.MESH,
            )
            r.start()
            kv_rdmas.append(r)

        qh = q_ref[pl.ds(base, HALF_ROWS)].astype(jnp.bfloat16)

        o_rdmas = []
        for c in range(NCHUNK):
            kv_rdmas[c].wait_recv()
            outs = []
            for j in range(RPC):
                k_all = jnp.concatenate([kh[c][j], kv_recv[c, 0, j]], axis=0)
                v_all = jnp.concatenate([vh[c][j], kv_recv[c, 1, j]], axis=0)
                s_i = lax.dot_general(
                    qh[2 * c + j], k_all, (((1,), (1,)), ((), ())),
                    preferred_element_type=jnp.float32,
                ) * scale
                p = jnp.exp(s_i)
                l = jnp.sum(p, axis=1, keepdims=True)
                o_i = lax.dot_general(
                    p.astype(jnp.bfloat16), v_all,
                    (((1,), (0,)), ((), ())),
                    preferred_element_type=jnp.float32,
                ) / l
                outs.append(o_i[None])
                o_send[c, j] = o_i.astype(jnp.bfloat16)
            out_ref[pl.ds(base + 2 * c, RPC)] = jnp.concatenate(outs, axis=0)
            r = pltpu.make_async_remote_copy(
                src_ref=o_send.at[c], dst_ref=o_recv.at[c],
                send_sem=o_ssem.at[c], recv_sem=o_rsem.at[c],
                device_id=x_nbr, device_id_type=pl.DeviceIdType.MESH,
            )
            r.start()
            o_rdmas.append(r)

        for c in range(NCHUNK):
            o_rdmas[c].wait_recv()
            out_ref[pl.ds(other + 2 * c, RPC)] = o_recv[c].astype(jnp.float32)

        for c in range(NCHUNK):
            kv_rdmas[c].wait_send()
            o_rdmas[c].wait_send()

    out_rows = pl.pallas_call(
        body,
        out_shape=jax.ShapeDtypeStruct((hb, s, d), jnp.float32),
        in_specs=[pl.BlockSpec(memory_space=pltpu.VMEM)] * 3,
        out_specs=pl.BlockSpec(memory_space=pltpu.VMEM),
        scratch_shapes=[
            pltpu.VMEM((NCHUNK, 2, RPC, s, d), jnp.bfloat16),
            pltpu.VMEM((NCHUNK, 2, RPC, s, d), jnp.bfloat16),
            pltpu.VMEM((NCHUNK, RPC, s, d), jnp.bfloat16),
            pltpu.VMEM((NCHUNK, RPC, s, d), jnp.bfloat16),
            pltpu.SemaphoreType.DMA((NCHUNK,)),
            pltpu.SemaphoreType.DMA((NCHUNK,)),
            pltpu.SemaphoreType.DMA((NCHUNK,)),
            pltpu.SemaphoreType.DMA((NCHUNK,)),
        ],
        compiler_params=pltpu.CompilerParams(collective_id=0),
    )(to_rows(Q), to_rows(K), to_rows(V))

    return jnp.transpose(out_rows.reshape(h, b, s, d), (1, 2, 0, 3))


# device time: 39216 ns/iter; 1.4823x vs baseline; 1.0498x over previous
import jax
import jax.numpy as jnp
from jax import lax
from jax.experimental import pallas as pl
from jax.experimental.pallas import tpu as pltpu

HALF_ROWS = 8
NCHUNK = 8


def kernel(Q, K, V):
    b, s, h, d = Q.shape
    hb = h * b
    scale = d ** -0.5

    def to_rows(A):
        return jnp.transpose(A, (2, 0, 1, 3)).reshape(hb, s, d)

    def body(q_ref, k_ref, v_ref, out_ref, kv_send, kv_recv, o_send, o_recv,
             kv_ssem, kv_rsem, o_ssem, o_rsem):
        my_x = lax.axis_index("x")
        my_y = lax.axis_index("y")
        y_nbr = (my_x, 1 - my_y)
        x_nbr = (1 - my_x, my_y)

        barrier = pltpu.get_barrier_semaphore()
        for nbr in (y_nbr, x_nbr):
            pl.semaphore_signal(
                barrier, inc=1, device_id=nbr,
                device_id_type=pl.DeviceIdType.MESH,
            )
        pl.semaphore_wait(barrier, 2)

        base = my_x * HALF_ROWS
        other = (1 - my_x) * HALF_ROWS

        kv_rdmas = []
        kh = []
        vh = []
        for c in range(NCHUNK):
            kc = k_ref[base + c].astype(jnp.bfloat16)
            vc = v_ref[base + c].astype(jnp.bfloat16)
            kh.append(kc)
            vh.append(vc)
            kv_send[c, 0] = kc
            kv_send[c, 1] = vc
            r = pltpu.make_async_remote_copy(
                src_ref=kv_send.at[c], dst_ref=kv_recv.at[c],
                send_sem=kv_ssem.at[c], recv_sem=kv_rsem.at[c],
                device_id=y_nbr, device_id_type=pl.DeviceIdType.MESH,
            )
            r.start()
            kv_rdmas.append(r)

        qh = q_ref[pl.ds(base, HALF_ROWS)].astype(jnp.bfloat16)

        o_rdmas = []
        for c in range(NCHUNK):
            kv_rdmas[c].wait_recv()
            k_all = jnp.concatenate([kh[c], kv_recv[c, 0]], axis=0)
            v_all = jnp.concatenate([vh[c], kv_recv[c, 1]], axis=0)
            s_i = lax.dot_general(
                qh[c], k_all, (((1,), (1,)), ((), ())),
                preferred_element_type=jnp.float32,
            ) * scale
            p = jnp.exp(s_i)
            l = jnp.sum(p, axis=1, keepdims=True)
            o_i = lax.dot_general(
                p.astype(jnp.bfloat16), v_all,
                (((1,), (0,)), ((), ())),
                preferred_element_type=jnp.float32,
            ) / l
            out_ref[pl.ds(base + c, 1)] = o_i[None]
            o_send[c] = o_i.astype(jnp.bfloat16)
            r = pltpu.make_async_remote_copy(
                src_ref=o_send.at[c], dst_ref=o_recv.at[c],
                send_sem=o_ssem.at[c], recv_sem=o_rsem.at[c],
                device_id=x_nbr, device_id_type=pl.DeviceIdType.MESH,
            )
            r.start()
            o_rdmas.append(r)

        for c in range(NCHUNK):
            o_rdmas[c].wait_recv()
            out_ref[pl.ds(other + c, 1)] = o_recv[c].astype(jnp.float32)[None]

        for c in range(NCHUNK):
            kv_rdmas[c].wait_send()
            o_rdmas[c].wait_send()

    out_rows = pl.pallas_call(
        body,
        out_shape=jax.ShapeDtypeStruct((hb, s, d), jnp.float32),
        in_specs=[pl.BlockSpec(memory_space=pltpu.VMEM)] * 3,
        out_specs=pl.BlockSpec(memory_space=pltpu.VMEM),
        scratch_shapes=[
            pltpu.VMEM((NCHUNK, 2, s, d), jnp.bfloat16),
            pltpu.VMEM((NCHUNK, 2, s, d), jnp.bfloat16),
            pltpu.VMEM((NCHUNK, s, d), jnp.bfloat16),
            pltpu.VMEM((NCHUNK, s, d), jnp.bfloat16),
            pltpu.SemaphoreType.DMA((NCHUNK,)),
            pltpu.SemaphoreType.DMA((NCHUNK,)),
            pltpu.SemaphoreType.DMA((NCHUNK,)),
            pltpu.SemaphoreType.DMA((NCHUNK,)),
        ],
        compiler_params=pltpu.CompilerParams(collective_id=0),
    )(to_rows(Q), to_rows(K), to_rows(V))

    return jnp.transpose(out_rows.reshape(h, b, s, d), (1, 2, 0, 3))
